# baseline (speedup 1.0000x reference)
import jax
import jax.numpy as jnp
from jax import lax
from jax.experimental import pallas as pl
from jax.experimental.pallas import tpu as pltpu
from jax.experimental.pallas import tpu_sc as plsc

N_USERS = 5000
N_NODES = 10000
N_PAD = 10240
D = 128
DH = 64
E = 320000
ALPHA = 0.1
BATCH = 16384

NC = 2
NS = 16
L = 16

EB = 128
NBUF = 4
CHUNK = 8
NCHUNK = 20
NBLK = CHUNK * NCHUNK
E_PAD = NS * NBLK * EB
ROWS_PER_TILE = N_PAD // NS
RO_BLK = 8


def _body(xt, ax0t, src_t, dst_t, w_t, users_t, items_t, out,
          xa, xb, src_v, dst_v, w_v, rows_v, uidx_v, iidx_v, gout_v,
          esem, gsems, ssems):
    c = lax.axis_index("c")
    s = lax.axis_index("s")
    r0 = s * ROWS_PER_TILE

    def start_chunk_load(ch):
        half = lax.rem(ch, 2)
        pltpu.async_copy(src_t.at[s, ch], src_v.at[half], esem)
        pltpu.async_copy(dst_t.at[s, ch], dst_v.at[half], esem)
        pltpu.async_copy(w_t.at[s, ch], w_v.at[half], esem)

    def wait_chunk_load():
        pltpu.make_async_copy(src_t.at[s, 0], src_v.at[0], esem).wait()
        pltpu.make_async_copy(dst_t.at[s, 0], dst_v.at[0], esem).wait()
        pltpu.make_async_copy(w_t.at[s, 0], w_v.at[0], esem).wait()

    def wait_gather(b):
        pltpu.make_async_copy(xt.at[c, pl.ds(0, EB)], rows_v.at[b],
                              gsems[b]).wait()

    def wait_scatter(xdst, b):
        pltpu.make_async_copy(rows_v.at[b], xdst.at[pl.ds(0, EB)],
                              ssems[b]).wait()

    pltpu.async_copy(xt.at[c, pl.ds(r0, ROWS_PER_TILE)],
                     xa.at[pl.ds(r0, ROWS_PER_TILE)], gsems[0])
    start_chunk_load(0)
    pltpu.make_async_copy(xt.at[c, pl.ds(r0, ROWS_PER_TILE)],
                          xa.at[pl.ds(r0, ROWS_PER_TILE)], gsems[0]).wait()

    def scale_block(rows_ref, w_ref, half, blk):
        @pl.loop(0, EB // L)
        def _(g):
            wg = w_ref[half, blk, pl.ds(g * L, L)]
            e0 = g * L
            for j in range(0, L, 4):
                ws = [wg[j + k] for k in range(4)]
                vals = [[rows_ref[e0 + j + k, pl.ds(q * L, L)]
                         for q in range(DH // L)] for k in range(4)]
                prods = [[v * ws[k] for v in vals[k]] for k in range(4)]
                for q in range(DH // L):
                    for k in range(4):
                        rows_ref[e0 + j + k, pl.ds(q * L, L)] = prods[k][q]

    def propagate(xcur, xnxt):
        pltpu.sync_copy(ax0t.at[c, pl.ds(r0, ROWS_PER_TILE)],
                        xnxt.at[pl.ds(r0, ROWS_PER_TILE)])
        plsc.subcore_barrier()

        @pl.loop(0, NCHUNK)
        def _(ch):
            half = lax.rem(ch, 2)
            wait_chunk_load()
            @pl.when(ch > 0)
            def _():
                for b in range(NBUF):
                    wait_scatter(xnxt, b)

            @pl.when(ch + 1 < NCHUNK)
            def _():
                start_chunk_load(ch + 1)

            for b in range(NBUF):
                pltpu.async_copy(xcur.at[src_v.at[half, b]], rows_v.at[b],
                                 gsems[b])
            for b in range(NBUF):
                wait_gather(b)
                scale_block(rows_v.at[b], w_v, half, b)
                pltpu.async_copy(rows_v.at[b], xnxt.at[dst_v.at[half, b]],
                                 ssems[b], add=True)
            for b in range(NBUF):
                wait_scatter(xnxt, b)
                pltpu.async_copy(xcur.at[src_v.at[half, NBUF + b]],
                                 rows_v.at[b], gsems[b])
            for b in range(NBUF):
                wait_gather(b)
                scale_block(rows_v.at[b], w_v, half, NBUF + b)
                pltpu.async_copy(rows_v.at[b],
                                 xnxt.at[dst_v.at[half, NBUF + b]],
                                 ssems[b], add=True)

        for b in range(NBUF):
            wait_scatter(xnxt, b)
        plsc.subcore_barrier()

    propagate(xa, xb)
    start_chunk_load(0)
    propagate(xb, xa)
    start_chunk_load(0)
    propagate(xa, xb)
    xfin = xb

    pltpu.sync_copy(users_t.at[s], uidx_v)
    pltpu.sync_copy(items_t.at[s], iidx_v)

    lane = lax.iota(jnp.int32, L)

    def ro_gather(r, pair):
        pltpu.async_copy(xfin.at[uidx_v.at[r]], rows_v.at[2 * pair],
                         gsems[2 * pair])
        pltpu.async_copy(xfin.at[iidx_v.at[r]], rows_v.at[2 * pair + 1],
                         gsems[2 * pair + 1])

    def ro_wait(r, pair):
        pltpu.make_async_copy(xfin.at[uidx_v.at[r]], rows_v.at[2 * pair],
                              gsems[2 * pair]).wait()
        pltpu.make_async_copy(xfin.at[iidx_v.at[r]], rows_v.at[2 * pair + 1],
                              gsems[2 * pair + 1]).wait()

    def ro_compute(r, pair):
        urows = rows_v.at[2 * pair]
        irows = rows_v.at[2 * pair + 1]

        @pl.loop(0, EB // L)
        def _(g):
            e0 = g * L
            dv = jnp.zeros((L,), jnp.float32)
            for j in range(L):
                acc = urows[e0 + j, pl.ds(0, L)] * irows[e0 + j, pl.ds(0, L)]
                for q in range(1, DH // L):
                    sl = pl.ds(q * L, L)
                    acc = acc + urows[e0 + j, sl] * irows[e0 + j, sl]
                dv = jnp.where(lane == j, plsc.cumsum(acc)[L - 1], dv)
            gout_v[r, pl.ds(g * L, L)] = dv

    ro_gather(0, 0)

    @pl.loop(0, RO_BLK, step=2)
    def _(r):
        ro_gather(r + 1, 1)
        ro_wait(r, 0)
        ro_compute(r, 0)

        @pl.when(r + 2 < RO_BLK)
        def _():
            ro_gather(r + 2, 0)

        ro_wait(r + 1, 1)
        ro_compute(r + 1, 1)

    pltpu.sync_copy(gout_v, out.at[c, s])


@jax.jit
def _run(xt, ax0t, src_t, dst_t, w_t, users_t, items_t):
    mesh = plsc.VectorSubcoreMesh(core_axis_name="c", subcore_axis_name="s")

    def body(xt, ax0t, src_t, dst_t, w_t, users_t, items_t, out,
             xa, xb, src_v, dst_v, w_v, rows_v, uidx_v, iidx_v, gout_v,
             esem, g0, g1, g2, g3, s0, s1, s2, s3):
        _body(xt, ax0t, src_t, dst_t, w_t, users_t, items_t, out,
              xa, xb, src_v, dst_v, w_v, rows_v, uidx_v, iidx_v, gout_v,
              esem, [g0, g1, g2, g3], [s0, s1, s2, s3])

    f = pl.kernel(
        body,
        out_type=jax.ShapeDtypeStruct((NC, NS, RO_BLK, EB), jnp.float32),
        mesh=mesh,
        compiler_params=pltpu.CompilerParams(needs_layout_passes=False,
                                             use_tc_tiling_on_sc=False),
        scratch_types=[
            pltpu.VMEM_SHARED((N_PAD, DH), jnp.float32),
            pltpu.VMEM_SHARED((N_PAD, DH), jnp.float32),
            pltpu.VMEM((2, CHUNK, EB), jnp.int32),
            pltpu.VMEM((2, CHUNK, EB), jnp.int32),
            pltpu.VMEM((2, CHUNK, EB), jnp.float32),
            pltpu.VMEM((NBUF, EB, DH), jnp.float32),
            pltpu.VMEM((RO_BLK, EB), jnp.int32),
            pltpu.VMEM((RO_BLK, EB), jnp.int32),
            pltpu.VMEM((RO_BLK, EB), jnp.float32),
            pltpu.SemaphoreType.DMA,
            pltpu.SemaphoreType.DMA,
            pltpu.SemaphoreType.DMA,
            pltpu.SemaphoreType.DMA,
            pltpu.SemaphoreType.DMA,
            pltpu.SemaphoreType.DMA,
            pltpu.SemaphoreType.DMA,
            pltpu.SemaphoreType.DMA,
            pltpu.SemaphoreType.DMA,
        ],
    )
    return f(xt, ax0t, src_t, dst_t, w_t, users_t, items_t)


def kernel(user_emb, item_emb, edge_weight, users, items, edge_index):
    x0 = jnp.concatenate([user_emb, item_emb], axis=0)
    xt = x0.reshape(N_NODES, NC, DH).transpose(1, 0, 2)
    xt = jnp.zeros((NC, N_PAD, DH), jnp.float32).at[:, :N_NODES].set(xt)
    ax0t = ALPHA * xt

    src = jnp.zeros((E_PAD,), jnp.int32).at[:E].set(edge_index[0])
    dst = jnp.zeros((E_PAD,), jnp.int32).at[:E].set(edge_index[1])
    w9 = jnp.zeros((E_PAD,), jnp.float32).at[:E].set(
        (1.0 - ALPHA) * edge_weight)
    src_t = src.reshape(NS, NCHUNK, CHUNK, EB)
    dst_t = dst.reshape(NS, NCHUNK, CHUNK, EB)
    w_t = w9.reshape(NS, NCHUNK, CHUNK, EB)

    users_t = users.reshape(NS, RO_BLK, EB)
    items_t = (items + N_USERS).astype(jnp.int32).reshape(NS, RO_BLK, EB)

    part = _run(xt, ax0t, src_t, dst_t, w_t, users_t, items_t)
    return part.reshape(NC, BATCH).sum(axis=0)


# speedup vs baseline: 1.0699x; 1.0699x over previous
import jax
import jax.numpy as jnp
from jax import lax
from jax.experimental import pallas as pl
from jax.experimental.pallas import tpu as pltpu
from jax.experimental.pallas import tpu_sc as plsc

N_USERS = 5000
N_NODES = 10000
N_PAD = 10240
D = 128
DH = 64
E = 320000
ALPHA = 0.1
BATCH = 16384

NC = 2
NS = 16
L = 16

EB = 128
NBUF = 4
CHUNK = 4
NCHUNK = 40
NBLK = CHUNK * NCHUNK
E_PAD = NS * NBLK * EB
ROWS_PER_TILE = N_PAD // NS
RO_BLK = 8


def _body(xt, ax0t, src_t, dst_t, w_t, users_t, items_t, out,
          xa, xb, src_v, dst_v, w_v, rows_v, uidx_v, iidx_v, gout_v,
          esem, gsems, ssems):
    c = lax.axis_index("c")
    s = lax.axis_index("s")
    r0 = s * ROWS_PER_TILE

    def start_chunk_load(ch):
        half = lax.rem(ch, 2)
        pltpu.async_copy(src_t.at[s, ch], src_v.at[half], esem)
        pltpu.async_copy(dst_t.at[s, ch], dst_v.at[half], esem)
        pltpu.async_copy(w_t.at[s, ch], w_v.at[half], esem)

    def wait_chunk_load():
        pltpu.make_async_copy(src_t.at[s, 0], src_v.at[0], esem).wait()
        pltpu.make_async_copy(dst_t.at[s, 0], dst_v.at[0], esem).wait()
        pltpu.make_async_copy(w_t.at[s, 0], w_v.at[0], esem).wait()

    def wait_gather(b):
        pltpu.make_async_copy(xt.at[c, pl.ds(0, EB)], rows_v.at[b],
                              gsems[b]).wait()

    def wait_scatter(xdst, b):
        pltpu.make_async_copy(rows_v.at[b], xdst.at[pl.ds(0, EB)],
                              ssems[b]).wait()

    pltpu.async_copy(xt.at[c, pl.ds(r0, ROWS_PER_TILE)],
                     xa.at[pl.ds(r0, ROWS_PER_TILE)], gsems[0])
    start_chunk_load(0)
    pltpu.make_async_copy(xt.at[c, pl.ds(r0, ROWS_PER_TILE)],
                          xa.at[pl.ds(r0, ROWS_PER_TILE)], gsems[0]).wait()

    def scale_block(rows_ref, w_ref, half, blk):
        @pl.loop(0, EB // L)
        def _(g):
            wg = w_ref[half, blk, pl.ds(g * L, L)]
            e0 = g * L
            for j in range(0, L, 4):
                ws = [wg[j + k] for k in range(4)]
                vals = [[rows_ref[e0 + j + k, pl.ds(q * L, L)]
                         for q in range(DH // L)] for k in range(4)]
                prods = [[v * ws[k] for v in vals[k]] for k in range(4)]
                for q in range(DH // L):
                    for k in range(4):
                        rows_ref[e0 + j + k, pl.ds(q * L, L)] = prods[k][q]

    def propagate(xcur, xnxt):
        pltpu.sync_copy(ax0t.at[c, pl.ds(r0, ROWS_PER_TILE)],
                        xnxt.at[pl.ds(r0, ROWS_PER_TILE)])
        plsc.subcore_barrier()

        @pl.loop(0, NCHUNK)
        def _(ch):
            half = lax.rem(ch, 2)
            wait_chunk_load()
            @pl.when(ch > 0)
            def _():
                for b in range(NBUF):
                    wait_scatter(xnxt, b)

            @pl.when(ch + 1 < NCHUNK)
            def _():
                start_chunk_load(ch + 1)

            for b in range(NBUF):
                pltpu.async_copy(xcur.at[src_v.at[half, b]], rows_v.at[b],
                                 gsems[b])
            for b in range(NBUF):
                wait_gather(b)
                scale_block(rows_v.at[b], w_v, half, b)
                pltpu.async_copy(rows_v.at[b], xnxt.at[dst_v.at[half, b]],
                                 ssems[b], add=True)


        for b in range(NBUF):
            wait_scatter(xnxt, b)
        plsc.subcore_barrier()

    propagate(xa, xb)
    start_chunk_load(0)
    propagate(xb, xa)
    start_chunk_load(0)
    propagate(xa, xb)
    xfin = xb

    pltpu.sync_copy(users_t.at[s], uidx_v)
    pltpu.sync_copy(items_t.at[s], iidx_v)

    lane = lax.iota(jnp.int32, L)

    def ro_gather(r, pair):
        pltpu.async_copy(xfin.at[uidx_v.at[r]], rows_v.at[2 * pair],
                         gsems[2 * pair])
        pltpu.async_copy(xfin.at[iidx_v.at[r]], rows_v.at[2 * pair + 1],
                         gsems[2 * pair + 1])

    def ro_wait(r, pair):
        pltpu.make_async_copy(xfin.at[uidx_v.at[r]], rows_v.at[2 * pair],
                              gsems[2 * pair]).wait()
        pltpu.make_async_copy(xfin.at[iidx_v.at[r]], rows_v.at[2 * pair + 1],
                              gsems[2 * pair + 1]).wait()

    def ro_compute(r, pair):
        urows = rows_v.at[2 * pair]
        irows = rows_v.at[2 * pair + 1]

        @pl.loop(0, EB // L)
        def _(g):
            e0 = g * L
            dv = jnp.zeros((L,), jnp.float32)
            for j in range(L):
                acc = urows[e0 + j, pl.ds(0, L)] * irows[e0 + j, pl.ds(0, L)]
                for q in range(1, DH // L):
                    sl = pl.ds(q * L, L)
                    acc = acc + urows[e0 + j, sl] * irows[e0 + j, sl]
                dv = jnp.where(lane == j, plsc.cumsum(acc)[L - 1], dv)
            gout_v[r, pl.ds(g * L, L)] = dv

    ro_gather(0, 0)

    @pl.loop(0, RO_BLK, step=2)
    def _(r):
        ro_gather(r + 1, 1)
        ro_wait(r, 0)
        ro_compute(r, 0)

        @pl.when(r + 2 < RO_BLK)
        def _():
            ro_gather(r + 2, 0)

        ro_wait(r + 1, 1)
        ro_compute(r + 1, 1)

    pltpu.sync_copy(gout_v, out.at[c, s])


@jax.jit
def _run(xt, ax0t, src_t, dst_t, w_t, users_t, items_t):
    mesh = plsc.VectorSubcoreMesh(core_axis_name="c", subcore_axis_name="s")

    def body(xt, ax0t, src_t, dst_t, w_t, users_t, items_t, out,
             xa, xb, src_v, dst_v, w_v, rows_v, uidx_v, iidx_v, gout_v,
             esem, g0, g1, g2, g3, s0, s1, s2, s3):
        _body(xt, ax0t, src_t, dst_t, w_t, users_t, items_t, out,
              xa, xb, src_v, dst_v, w_v, rows_v, uidx_v, iidx_v, gout_v,
              esem, [g0, g1, g2, g3], [s0, s1, s2, s3])

    f = pl.kernel(
        body,
        out_type=jax.ShapeDtypeStruct((NC, NS, RO_BLK, EB), jnp.float32),
        mesh=mesh,
        compiler_params=pltpu.CompilerParams(needs_layout_passes=False,
                                             use_tc_tiling_on_sc=False),
        scratch_types=[
            pltpu.VMEM_SHARED((N_PAD, DH), jnp.float32),
            pltpu.VMEM_SHARED((N_PAD, DH), jnp.float32),
            pltpu.VMEM((2, CHUNK, EB), jnp.int32),
            pltpu.VMEM((2, CHUNK, EB), jnp.int32),
            pltpu.VMEM((2, CHUNK, EB), jnp.float32),
            pltpu.VMEM((NBUF, EB, DH), jnp.float32),
            pltpu.VMEM((RO_BLK, EB), jnp.int32),
            pltpu.VMEM((RO_BLK, EB), jnp.int32),
            pltpu.VMEM((RO_BLK, EB), jnp.float32),
            pltpu.SemaphoreType.DMA,
            pltpu.SemaphoreType.DMA,
            pltpu.SemaphoreType.DMA,
            pltpu.SemaphoreType.DMA,
            pltpu.SemaphoreType.DMA,
            pltpu.SemaphoreType.DMA,
            pltpu.SemaphoreType.DMA,
            pltpu.SemaphoreType.DMA,
            pltpu.SemaphoreType.DMA,
        ],
    )
    return f(xt, ax0t, src_t, dst_t, w_t, users_t, items_t)


def kernel(user_emb, item_emb, edge_weight, users, items, edge_index):
    x0 = jnp.concatenate([user_emb, item_emb], axis=0)
    xt = x0.reshape(N_NODES, NC, DH).transpose(1, 0, 2)
    xt = jnp.zeros((NC, N_PAD, DH), jnp.float32).at[:, :N_NODES].set(xt)
    ax0t = ALPHA * xt

    src = jnp.zeros((E_PAD,), jnp.int32).at[:E].set(edge_index[0])
    dst = jnp.zeros((E_PAD,), jnp.int32).at[:E].set(edge_index[1])
    w9 = jnp.zeros((E_PAD,), jnp.float32).at[:E].set(
        (1.0 - ALPHA) * edge_weight)
    src_t = src.reshape(NS, NCHUNK, CHUNK, EB)
    dst_t = dst.reshape(NS, NCHUNK, CHUNK, EB)
    w_t = w9.reshape(NS, NCHUNK, CHUNK, EB)

    users_t = users.reshape(NS, RO_BLK, EB)
    items_t = (items + N_USERS).astype(jnp.int32).reshape(NS, RO_BLK, EB)

    part = _run(xt, ax0t, src_t, dst_t, w_t, users_t, items_t)
    return part.reshape(NC, BATCH).sum(axis=0)
